# baseline (speedup 1.0000x reference)
import functools

import jax
import jax.numpy as jnp
from jax import lax
from jax.experimental import pallas as pl
from jax.experimental.pallas import tpu as pltpu
from jax.experimental.pallas import tpu_sc as plsc

N_NODES = 10000
N_EDGES = 320000
NUM_KERNELS = 4
NUM_POWERS = 3
HID = 64
NUM_GRAPHS = 16
KMEANS_MAX_ITER = 300
KMEANS_TOL = 1e-4

EDGE_CHUNK = 6400

NC = 2
NS = 16
L = 16
NW = NC * NS
EPW = N_EDGES // NW
ESTEPS = EPW // L


def _sc_mesh():
    return plsc.VectorSubcoreMesh(core_axis_name="c", subcore_axis_name="s")


def _sc_edge_gather_body(v_hbm, src_hbm, dst_hbm, ones_hbm, zeros_hbm,
                         a_hbm, s_hbm, degp_hbm,
                         sidx, didx, abuf, sbuf, ones, degsh, sem):
    c = lax.axis_index("c")
    s = lax.axis_index("s")
    wid = s * NC + c
    base = wid * EPW
    pltpu.sync_copy(src_hbm.at[pl.ds(base, EPW)], sidx)
    pltpu.sync_copy(dst_hbm.at[pl.ds(base, EPW)], didx)
    pltpu.sync_copy(ones_hbm, ones)

    @pl.when(s == 0)
    def _():
        pltpu.sync_copy(zeros_hbm, degsh)
    plsc.subcore_barrier()

    pltpu.async_copy(v_hbm.at[didx], abuf, sem).wait()
    pltpu.async_copy(v_hbm.at[sidx], sbuf, sem).wait()
    pltpu.sync_copy(abuf, a_hbm.at[pl.ds(base, EPW)])
    pltpu.sync_copy(sbuf, s_hbm.at[pl.ds(base, EPW)])

    pltpu.sync_copy(ones, degsh.at[sidx], add=True)
    plsc.subcore_barrier()

    @pl.when(s == 0)
    def _():
        pltpu.sync_copy(degsh, degp_hbm.at[c])


def _sc_edge_gather(v, src, dst):
    ones = jnp.ones((EPW,), jnp.float32)
    zeros = jnp.zeros((N_NODES,), jnp.float32)
    kfn = functools.partial(
        pl.kernel,
        mesh=_sc_mesh(),
        out_type=[
            jax.ShapeDtypeStruct((N_EDGES,), jnp.float32),
            jax.ShapeDtypeStruct((N_EDGES,), jnp.float32),
            jax.ShapeDtypeStruct((NC, N_NODES), jnp.float32),
        ],
        scratch_types=[
            pltpu.VMEM((EPW,), jnp.int32),
            pltpu.VMEM((EPW,), jnp.int32),
            pltpu.VMEM((EPW,), jnp.float32),
            pltpu.VMEM((EPW,), jnp.float32),
            pltpu.VMEM((EPW,), jnp.float32),
            pltpu.VMEM_SHARED((N_NODES,), jnp.float32),
            pltpu.SemaphoreType.DMA,
        ],
    )
    return kfn(_sc_edge_gather_body)(v, src, dst, ones, zeros)


def _sc_edge_combine_body(q_hbm, assign_hbm, deg_hbm, src_hbm, dst_hbm, zeros_hbm,
                          outp_hbm,
                          sidx, didx, q0b, q1b, q2b, q3b, ksb, kdb, degb, nbuf,
                          accsh, sem):
    c = lax.axis_index("c")
    s = lax.axis_index("s")
    wid = s * NC + c
    base = wid * EPW
    pltpu.sync_copy(src_hbm.at[pl.ds(base, EPW)], sidx)
    pltpu.sync_copy(dst_hbm.at[pl.ds(base, EPW)], didx)
    pltpu.sync_copy(q_hbm.at[0].at[wid], q0b)
    pltpu.sync_copy(q_hbm.at[1].at[wid], q1b)
    pltpu.sync_copy(q_hbm.at[2].at[wid], q2b)
    pltpu.sync_copy(q_hbm.at[3].at[wid], q3b)

    @pl.when(s == 0)
    def _():
        pltpu.sync_copy(zeros_hbm, accsh)

    pltpu.async_copy(assign_hbm.at[sidx], ksb, sem).wait()
    pltpu.async_copy(assign_hbm.at[didx], kdb, sem).wait()
    pltpu.async_copy(deg_hbm.at[sidx], degb, sem).wait()

    def step(i, _):
        sl = pl.ds(i * L, L)
        ks = ksb[sl]
        kd = kdb[sl]
        degs = degb[sl]
        q = q0b[sl]
        q = jnp.where(ks == 1, q1b[sl], q)
        q = jnp.where(ks == 2, q2b[sl], q)
        q = jnp.where(ks == 3, q3b[sl], q)
        combined = jnp.where(ks == kd, q, jnp.zeros((L,), jnp.float32))
        nbuf[sl] = combined / degs
        return 0

    lax.fori_loop(0, ESTEPS, step, 0)

    plsc.subcore_barrier()
    pltpu.sync_copy(nbuf, accsh.at[didx], add=True)
    plsc.subcore_barrier()

    @pl.when(s == 0)
    def _():
        pltpu.sync_copy(accsh, outp_hbm.at[c])


def _sc_edge_combine(q4, assignments, deg, src, dst):
    zeros = jnp.zeros((N_NODES,), jnp.float32)
    kfn = functools.partial(
        pl.kernel,
        mesh=_sc_mesh(),
        out_type=jax.ShapeDtypeStruct((NC, N_NODES), jnp.float32),
        scratch_types=[
            pltpu.VMEM((EPW,), jnp.int32),
            pltpu.VMEM((EPW,), jnp.int32),
            pltpu.VMEM((EPW,), jnp.float32),
            pltpu.VMEM((EPW,), jnp.float32),
            pltpu.VMEM((EPW,), jnp.float32),
            pltpu.VMEM((EPW,), jnp.float32),
            pltpu.VMEM((EPW,), jnp.int32),
            pltpu.VMEM((EPW,), jnp.int32),
            pltpu.VMEM((EPW,), jnp.float32),
            pltpu.VMEM((EPW,), jnp.float32),
            pltpu.VMEM_SHARED((N_NODES,), jnp.float32),
            pltpu.SemaphoreType.DMA,
        ],
    )
    return kfn(_sc_edge_combine_body)(q4, assignments, deg, src, dst, zeros)


def _edge_mlp_body(a_ref, s_ref, W1_ref, b1_ref, W2_ref, b2_ref, eat_ref,
                   cw_ref, Aq_ref, out_ref, q_ref):
    a = a_ref[...]
    b = s_ref[...] - a
    W1 = W1_ref[...]
    h = a * W1[0:1, :] + b * W1[1:2, :] + b1_ref[...]
    h = jnp.maximum(h, 0.0)
    h = jnp.dot(h, W2_ref[...], preferred_element_type=jnp.float32) + b2_ref[...]
    out_ref[...] = jnp.maximum(h, 0.0)
    p = jnp.dot(cw_ref[...], eat_ref[...], preferred_element_type=jnp.float32)
    p1 = p[1:2, :]
    p2 = p[2:3, :]
    l1 = jnp.where(p1 >= 0.0, p1, 0.1 * p1)
    l2 = jnp.where(p2 >= 0.0, p2, 0.1 * p2)
    lmat = jnp.concatenate([p[0:1, :], l1, l2 * l2], axis=0)
    q_ref[...] = jnp.dot(Aq_ref[...], lmat, preferred_element_type=jnp.float32)


def _edge_mlp(a, s, W1, b1, W2, b2, eat, cw, Aq):
    grid = (N_EDGES // EDGE_CHUNK,)
    return pl.pallas_call(
        _edge_mlp_body,
        grid=grid,
        in_specs=[
            pl.BlockSpec((EDGE_CHUNK, 1), lambda i: (i, 0)),
            pl.BlockSpec((EDGE_CHUNK, 1), lambda i: (i, 0)),
            pl.BlockSpec((2, HID), lambda i: (0, 0)),
            pl.BlockSpec((1, HID), lambda i: (0, 0)),
            pl.BlockSpec((HID, HID), lambda i: (0, 0)),
            pl.BlockSpec((1, HID), lambda i: (0, 0)),
            pl.BlockSpec((2, EDGE_CHUNK), lambda i: (0, i)),
            pl.BlockSpec((NUM_POWERS, 2), lambda i: (0, 0)),
            pl.BlockSpec((NUM_KERNELS, NUM_POWERS), lambda i: (0, 0)),
        ],
        out_specs=[
            pl.BlockSpec((EDGE_CHUNK, HID), lambda i: (i, 0)),
            pl.BlockSpec((NUM_KERNELS, EDGE_CHUNK), lambda i: (0, i)),
        ],
        out_shape=[
            jax.ShapeDtypeStruct((N_EDGES, HID), jnp.float32),
            jax.ShapeDtypeStruct((NUM_KERNELS, N_EDGES), jnp.float32),
        ],
    )(a, s, W1, b1, W2, b2, eat, cw, Aq)


def _kmeans_body(x_ref, xt_ref, c0_ref, out_ref):
    X = x_ref[...]
    XT = xt_ref[...]
    x2t = jnp.sum(XT * XT, axis=0, keepdims=True)

    def cond_fn(state):
        _c, i, done = state
        return (i < KMEANS_MAX_ITER) & jnp.logical_not(done)

    def body_fn(state):
        C, i, _done = state
        c2 = jnp.sum(C * C, axis=1, keepdims=True)
        XCt = jnp.dot(C, XT, preferred_element_type=jnp.float32)
        d2 = jnp.maximum(x2t + c2 - 2.0 * XCt, 0.0)
        bv = d2[0:1, :]
        bi = jnp.zeros((1, N_NODES), jnp.int32)
        for j in range(1, NUM_KERNELS):
            take = d2[j:j + 1, :] < bv
            bv = jnp.where(take, d2[j:j + 1, :], bv)
            bi = jnp.where(take, j, bi)
        out_ref[...] = bi
        onehot = (bi == lax.broadcasted_iota(jnp.int32, (NUM_KERNELS, 1), 0)).astype(jnp.float32)
        sums = jnp.dot(onehot, X, preferred_element_type=jnp.float32)
        counts = jnp.sum(onehot, axis=1, keepdims=True)
        new_c = sums / jnp.maximum(counts, 1.0)
        done = jnp.sqrt(jnp.sum((new_c - C) ** 2)) < KMEANS_TOL
        C = jnp.where(done, C, new_c)
        return C, i + 1, done

    C0 = c0_ref[...]
    state = (C0, jnp.int32(0), jnp.bool_(False))
    lax.while_loop(cond_fn, body_fn, state)


def _kmeans(X, key):
    init_idx = jax.random.randint(key, (NUM_KERNELS,), 0, X.shape[0])
    c0 = X[init_idx]
    out = pl.pallas_call(
        _kmeans_body,
        out_shape=jax.ShapeDtypeStruct((1, N_NODES), jnp.int32),
    )(X, X.T, c0)
    return out[0]


def kernel(x, edge_index, edge_attr, batch, W_similar, b_similar, W_ec1, b_ec1,
           W_ec2, b_ec2, conv_w, alpha, W_fc, b_fc):
    src = edge_index[0]
    dst = edge_index[1]

    v = jnp.maximum(x @ (W_similar[:2] + W_similar[2:]) + b_similar, 0.0)

    a, svals, degp = _sc_edge_gather(v[:, 0], src, dst)
    deg = degp[0] + degp[1]

    Aq = alpha[:, :, 0, 0]
    h, q4 = _edge_mlp(a[:, None], svals[:, None], W_ec1, b_ec1.reshape(1, HID),
                      W_ec2, b_ec2.reshape(1, HID), edge_attr.T,
                      conv_w[:, :, 0], Aq)
    x_sim = jnp.zeros((N_NODES, HID), jnp.float32).at[dst].max(
        h, mode="promise_in_bounds", unique_indices=False, indices_are_sorted=False)

    assignments = _kmeans(x_sim, jax.random.key(42))

    outp = _sc_edge_combine(q4.reshape(NUM_KERNELS, NW, EPW), assignments, deg, src, dst)
    node_out = outp[0] + outp[1]

    h5 = jnp.maximum(node_out, 0.0)[:, None]
    sums = jax.ops.segment_sum(h5, batch, num_segments=NUM_GRAPHS)
    counts = jax.ops.segment_sum(jnp.ones((N_NODES,), jnp.float32), batch, num_segments=NUM_GRAPHS)
    pooled = sums / jnp.maximum(counts, 1.0)[:, None]
    return pooled @ W_fc + b_fc


# speedup vs baseline: 1.0318x; 1.0318x over previous
import functools

import jax
import jax.numpy as jnp
from jax import lax
from jax.experimental import pallas as pl
from jax.experimental.pallas import tpu as pltpu
from jax.experimental.pallas import tpu_sc as plsc

N_NODES = 10000
N_EDGES = 320000
NUM_KERNELS = 4
NUM_POWERS = 3
HID = 64
NUM_GRAPHS = 16
KMEANS_MAX_ITER = 300
KMEANS_TOL = 1e-4

EDGE_CHUNK = 6400

NC = 2
NS = 16
L = 16
NW = NC * NS
EPW = N_EDGES // NW
ESTEPS = EPW // L


def _sc_mesh():
    return plsc.VectorSubcoreMesh(core_axis_name="c", subcore_axis_name="s")


def _sc_edge_gather_body(v_hbm, src_hbm, dst_hbm, ones_hbm, zeros_hbm,
                         a_hbm, s_hbm, degp_hbm,
                         sidx, didx, abuf, sbuf, ones, degsh, sem):
    c = lax.axis_index("c")
    s = lax.axis_index("s")
    wid = s * NC + c
    base = wid * EPW
    pltpu.sync_copy(src_hbm.at[pl.ds(base, EPW)], sidx)
    pltpu.sync_copy(dst_hbm.at[pl.ds(base, EPW)], didx)
    pltpu.sync_copy(ones_hbm, ones)

    @pl.when(s == 0)
    def _():
        pltpu.sync_copy(zeros_hbm, degsh)
    plsc.subcore_barrier()

    pltpu.async_copy(v_hbm.at[didx], abuf, sem).wait()
    pltpu.async_copy(v_hbm.at[sidx], sbuf, sem).wait()
    pltpu.sync_copy(abuf, a_hbm.at[pl.ds(base, EPW)])
    pltpu.sync_copy(sbuf, s_hbm.at[pl.ds(base, EPW)])

    pltpu.sync_copy(ones, degsh.at[sidx], add=True)
    plsc.subcore_barrier()

    @pl.when(s == 0)
    def _():
        pltpu.sync_copy(degsh, degp_hbm.at[c])


def _sc_edge_gather(v, src, dst):
    ones = jnp.ones((EPW,), jnp.float32)
    zeros = jnp.zeros((N_NODES,), jnp.float32)
    kfn = functools.partial(
        pl.kernel,
        mesh=_sc_mesh(),
        out_type=[
            jax.ShapeDtypeStruct((N_EDGES,), jnp.float32),
            jax.ShapeDtypeStruct((N_EDGES,), jnp.float32),
            jax.ShapeDtypeStruct((NC, N_NODES), jnp.float32),
        ],
        scratch_types=[
            pltpu.VMEM((EPW,), jnp.int32),
            pltpu.VMEM((EPW,), jnp.int32),
            pltpu.VMEM((EPW,), jnp.float32),
            pltpu.VMEM((EPW,), jnp.float32),
            pltpu.VMEM((EPW,), jnp.float32),
            pltpu.VMEM_SHARED((N_NODES,), jnp.float32),
            pltpu.SemaphoreType.DMA,
        ],
    )
    return kfn(_sc_edge_gather_body)(v, src, dst, ones, zeros)


def _sc_edge_combine_body(q_hbm, assign_hbm, deg_hbm, src_hbm, dst_hbm, zeros_hbm,
                          outp_hbm,
                          sidx, didx, q0b, q1b, q2b, q3b, ksb, kdb, degb, nbuf,
                          accsh, sem):
    c = lax.axis_index("c")
    s = lax.axis_index("s")
    wid = s * NC + c
    base = wid * EPW
    pltpu.sync_copy(src_hbm.at[pl.ds(base, EPW)], sidx)
    pltpu.sync_copy(dst_hbm.at[pl.ds(base, EPW)], didx)
    pltpu.sync_copy(q_hbm.at[0].at[wid], q0b)
    pltpu.sync_copy(q_hbm.at[1].at[wid], q1b)
    pltpu.sync_copy(q_hbm.at[2].at[wid], q2b)
    pltpu.sync_copy(q_hbm.at[3].at[wid], q3b)

    @pl.when(s == 0)
    def _():
        pltpu.sync_copy(zeros_hbm, accsh)

    pltpu.async_copy(assign_hbm.at[sidx], ksb, sem).wait()
    pltpu.async_copy(assign_hbm.at[didx], kdb, sem).wait()
    pltpu.async_copy(deg_hbm.at[sidx], degb, sem).wait()

    def step(i, _):
        sl = pl.ds(i * L, L)
        ks = ksb[sl]
        kd = kdb[sl]
        degs = degb[sl]
        q = q0b[sl]
        q = jnp.where(ks == 1, q1b[sl], q)
        q = jnp.where(ks == 2, q2b[sl], q)
        q = jnp.where(ks == 3, q3b[sl], q)
        combined = jnp.where(ks == kd, q, jnp.zeros((L,), jnp.float32))
        nbuf[sl] = combined / degs
        return 0

    lax.fori_loop(0, ESTEPS, step, 0)

    plsc.subcore_barrier()
    pltpu.sync_copy(nbuf, accsh.at[didx], add=True)
    plsc.subcore_barrier()

    @pl.when(s == 0)
    def _():
        pltpu.sync_copy(accsh, outp_hbm.at[c])


def _sc_edge_combine(q4, assignments, deg, src, dst):
    zeros = jnp.zeros((N_NODES,), jnp.float32)
    kfn = functools.partial(
        pl.kernel,
        mesh=_sc_mesh(),
        out_type=jax.ShapeDtypeStruct((NC, N_NODES), jnp.float32),
        scratch_types=[
            pltpu.VMEM((EPW,), jnp.int32),
            pltpu.VMEM((EPW,), jnp.int32),
            pltpu.VMEM((EPW,), jnp.float32),
            pltpu.VMEM((EPW,), jnp.float32),
            pltpu.VMEM((EPW,), jnp.float32),
            pltpu.VMEM((EPW,), jnp.float32),
            pltpu.VMEM((EPW,), jnp.int32),
            pltpu.VMEM((EPW,), jnp.int32),
            pltpu.VMEM((EPW,), jnp.float32),
            pltpu.VMEM((EPW,), jnp.float32),
            pltpu.VMEM_SHARED((N_NODES,), jnp.float32),
            pltpu.SemaphoreType.DMA,
        ],
    )
    return kfn(_sc_edge_combine_body)(q4, assignments, deg, src, dst, zeros)


def _edge_mlp_body(a_ref, s_ref, W1_ref, b1_ref, W2_ref, b2_ref, eat_ref,
                   cw_ref, Aq_ref, out_ref, q_ref):
    a = a_ref[...]
    b = s_ref[...] - a
    W1 = W1_ref[...]
    h = a * W1[0:1, :] + b * W1[1:2, :] + b1_ref[...]
    h = jnp.maximum(h, 0.0)
    h = jnp.dot(h, W2_ref[...], preferred_element_type=jnp.float32) + b2_ref[...]
    out_ref[...] = jnp.maximum(h, 0.0)
    p = jnp.dot(cw_ref[...], eat_ref[...], preferred_element_type=jnp.float32)
    p1 = p[1:2, :]
    p2 = p[2:3, :]
    l1 = jnp.where(p1 >= 0.0, p1, 0.1 * p1)
    l2 = jnp.where(p2 >= 0.0, p2, 0.1 * p2)
    lmat = jnp.concatenate([p[0:1, :], l1, l2 * l2], axis=0)
    q_ref[...] = jnp.dot(Aq_ref[...], lmat, preferred_element_type=jnp.float32)


def _edge_mlp(a, s, W1, b1, W2, b2, eat, cw, Aq):
    grid = (N_EDGES // EDGE_CHUNK,)
    return pl.pallas_call(
        _edge_mlp_body,
        grid=grid,
        in_specs=[
            pl.BlockSpec((EDGE_CHUNK, 1), lambda i: (i, 0)),
            pl.BlockSpec((EDGE_CHUNK, 1), lambda i: (i, 0)),
            pl.BlockSpec((2, HID), lambda i: (0, 0)),
            pl.BlockSpec((1, HID), lambda i: (0, 0)),
            pl.BlockSpec((HID, HID), lambda i: (0, 0)),
            pl.BlockSpec((1, HID), lambda i: (0, 0)),
            pl.BlockSpec((2, EDGE_CHUNK), lambda i: (0, i)),
            pl.BlockSpec((NUM_POWERS, 2), lambda i: (0, 0)),
            pl.BlockSpec((NUM_KERNELS, NUM_POWERS), lambda i: (0, 0)),
        ],
        out_specs=[
            pl.BlockSpec((EDGE_CHUNK, HID), lambda i: (i, 0)),
            pl.BlockSpec((NUM_KERNELS, EDGE_CHUNK), lambda i: (0, i)),
        ],
        out_shape=[
            jax.ShapeDtypeStruct((N_EDGES, HID), jnp.float32),
            jax.ShapeDtypeStruct((NUM_KERNELS, N_EDGES), jnp.float32),
        ],
    )(a, s, W1, b1, W2, b2, eat, cw, Aq)


def _pool_body(outp_ref, batch_ref, Wfc_ref, bfc_ref, out_ref):
    no = outp_ref[0:1, :] + outp_ref[1:2, :]
    h5 = jnp.maximum(no, 0.0)
    gid = lax.broadcasted_iota(jnp.int32, (NUM_GRAPHS, 1), 0)
    oh = (batch_ref[...] == gid).astype(jnp.float32)
    sums = jnp.sum(oh * h5, axis=1, keepdims=True)
    counts = jnp.sum(oh, axis=1, keepdims=True)
    pooled = sums / jnp.maximum(counts, 1.0)
    out_ref[...] = pooled * Wfc_ref[0, 0] + bfc_ref[0, 0]


def _pool(outp, batch, W_fc, b_fc):
    return pl.pallas_call(
        _pool_body,
        out_shape=jax.ShapeDtypeStruct((NUM_GRAPHS, 1), jnp.float32),
    )(outp, batch.reshape(1, N_NODES), W_fc, b_fc.reshape(1, 1))


def _kmeans_body(x_ref, xt_ref, c0_ref, out_ref):
    X = x_ref[...]
    XT = xt_ref[...]
    x2t = jnp.sum(XT * XT, axis=0, keepdims=True)

    def cond_fn(state):
        _c, i, done = state
        return (i < KMEANS_MAX_ITER) & jnp.logical_not(done)

    def body_fn(state):
        C, i, _done = state
        c2 = jnp.sum(C * C, axis=1, keepdims=True)
        XCt = jnp.dot(C, XT, preferred_element_type=jnp.float32)
        d2 = jnp.maximum(x2t + c2 - 2.0 * XCt, 0.0)
        bv = d2[0:1, :]
        bi = jnp.zeros((1, N_NODES), jnp.int32)
        for j in range(1, NUM_KERNELS):
            take = d2[j:j + 1, :] < bv
            bv = jnp.where(take, d2[j:j + 1, :], bv)
            bi = jnp.where(take, j, bi)
        out_ref[...] = bi
        onehot = (bi == lax.broadcasted_iota(jnp.int32, (NUM_KERNELS, 1), 0)).astype(jnp.float32)
        sums = jnp.dot(onehot, X, preferred_element_type=jnp.float32)
        counts = jnp.sum(onehot, axis=1, keepdims=True)
        new_c = sums / jnp.maximum(counts, 1.0)
        done = jnp.sqrt(jnp.sum((new_c - C) ** 2)) < KMEANS_TOL
        C = jnp.where(done, C, new_c)
        return C, i + 1, done

    C0 = c0_ref[...]
    state = (C0, jnp.int32(0), jnp.bool_(False))
    lax.while_loop(cond_fn, body_fn, state)


def _kmeans(X, key):
    init_idx = jax.random.randint(key, (NUM_KERNELS,), 0, X.shape[0])
    c0 = X[init_idx]
    out = pl.pallas_call(
        _kmeans_body,
        out_shape=jax.ShapeDtypeStruct((1, N_NODES), jnp.int32),
    )(X, X.T, c0)
    return out[0]


def kernel(x, edge_index, edge_attr, batch, W_similar, b_similar, W_ec1, b_ec1,
           W_ec2, b_ec2, conv_w, alpha, W_fc, b_fc):
    src = edge_index[0]
    dst = edge_index[1]

    v = jnp.maximum(x @ (W_similar[:2] + W_similar[2:]) + b_similar, 0.0)

    a, svals, degp = _sc_edge_gather(v[:, 0], src, dst)
    deg = degp[0] + degp[1]

    Aq = alpha[:, :, 0, 0]
    h, q4 = _edge_mlp(a[:, None], svals[:, None], W_ec1, b_ec1.reshape(1, HID),
                      W_ec2, b_ec2.reshape(1, HID), edge_attr.T,
                      conv_w[:, :, 0], Aq)
    x_sim = jnp.zeros((N_NODES, HID), jnp.float32).at[dst].max(
        h, mode="promise_in_bounds", unique_indices=False, indices_are_sorted=False)

    assignments = _kmeans(x_sim, jax.random.key(42))

    outp = _sc_edge_combine(q4.reshape(NUM_KERNELS, NW, EPW), assignments, deg, src, dst)

    return _pool(outp, batch, W_fc, b_fc)


# speedup vs baseline: 1.0937x; 1.0601x over previous
import functools

import jax
import jax.numpy as jnp
from jax import lax
from jax.experimental import pallas as pl
from jax.experimental.pallas import tpu as pltpu
from jax.experimental.pallas import tpu_sc as plsc

N_NODES = 10000
N_EDGES = 320000
NUM_KERNELS = 4
NUM_POWERS = 3
HID = 64
NUM_GRAPHS = 16
KMEANS_MAX_ITER = 300
KMEANS_TOL = 1e-4

EDGE_CHUNK = 6400

NC = 2
NS = 16
L = 16
NW = NC * NS
EPW = N_EDGES // NW
ESTEPS = EPW // L


def _sc_mesh():
    return plsc.VectorSubcoreMesh(core_axis_name="c", subcore_axis_name="s")


def _sc_edge_gather_body(v_hbm, src_hbm, dst_hbm, ones_hbm, zeros_hbm,
                         a_hbm, s_hbm, degp_hbm,
                         sidx, didx, abuf, sbuf, ones, degsh, sem):
    c = lax.axis_index("c")
    s = lax.axis_index("s")
    wid = s * NC + c
    base = wid * EPW
    pltpu.sync_copy(src_hbm.at[pl.ds(base, EPW)], sidx)
    pltpu.sync_copy(dst_hbm.at[pl.ds(base, EPW)], didx)
    pltpu.sync_copy(ones_hbm, ones)

    @pl.when(s == 0)
    def _():
        pltpu.sync_copy(zeros_hbm, degsh)
    plsc.subcore_barrier()

    pltpu.async_copy(v_hbm.at[didx], abuf, sem).wait()
    pltpu.async_copy(v_hbm.at[sidx], sbuf, sem).wait()
    pltpu.sync_copy(abuf, a_hbm.at[pl.ds(base, EPW)])
    pltpu.sync_copy(sbuf, s_hbm.at[pl.ds(base, EPW)])

    pltpu.sync_copy(ones, degsh.at[sidx], add=True)
    plsc.subcore_barrier()

    @pl.when(s == 0)
    def _():
        pltpu.sync_copy(degsh, degp_hbm.at[c])


def _sc_edge_gather(v, src, dst):
    ones = jnp.ones((EPW,), jnp.float32)
    zeros = jnp.zeros((N_NODES,), jnp.float32)
    kfn = functools.partial(
        pl.kernel,
        mesh=_sc_mesh(),
        out_type=[
            jax.ShapeDtypeStruct((N_EDGES,), jnp.float32),
            jax.ShapeDtypeStruct((N_EDGES,), jnp.float32),
            jax.ShapeDtypeStruct((NC, N_NODES), jnp.float32),
        ],
        scratch_types=[
            pltpu.VMEM((EPW,), jnp.int32),
            pltpu.VMEM((EPW,), jnp.int32),
            pltpu.VMEM((EPW,), jnp.float32),
            pltpu.VMEM((EPW,), jnp.float32),
            pltpu.VMEM((EPW,), jnp.float32),
            pltpu.VMEM_SHARED((N_NODES,), jnp.float32),
            pltpu.SemaphoreType.DMA,
        ],
    )
    return kfn(_sc_edge_gather_body)(v, src, dst, ones, zeros)


def _sc_edge_combine_body(q_hbm, assign_hbm, deg_hbm, src_hbm, dst_hbm, zeros_hbm,
                          outp_hbm,
                          sidx, didx, q0b, q1b, q2b, q3b, ksb, kdb, degb, nbuf,
                          accsh, assignsh, degsh, sem):
    c = lax.axis_index("c")
    s = lax.axis_index("s")
    wid = s * NC + c
    base = wid * EPW
    pltpu.sync_copy(src_hbm.at[pl.ds(base, EPW)], sidx)
    pltpu.sync_copy(dst_hbm.at[pl.ds(base, EPW)], didx)
    pltpu.sync_copy(q_hbm.at[0].at[wid], q0b)
    pltpu.sync_copy(q_hbm.at[1].at[wid], q1b)
    pltpu.sync_copy(q_hbm.at[2].at[wid], q2b)
    pltpu.sync_copy(q_hbm.at[3].at[wid], q3b)

    @pl.when(s == 0)
    def _():
        pltpu.sync_copy(zeros_hbm, accsh)
        pltpu.sync_copy(assign_hbm, assignsh)
        pltpu.sync_copy(deg_hbm, degsh)
    plsc.subcore_barrier()

    pltpu.async_copy(assignsh.at[sidx], ksb, sem).wait()
    pltpu.async_copy(assignsh.at[didx], kdb, sem).wait()
    pltpu.async_copy(degsh.at[sidx], degb, sem).wait()

    def step(i, _):
        sl = pl.ds(i * L, L)
        ks = ksb[sl]
        kd = kdb[sl]
        degs = degb[sl]
        q = q0b[sl]
        q = jnp.where(ks == 1, q1b[sl], q)
        q = jnp.where(ks == 2, q2b[sl], q)
        q = jnp.where(ks == 3, q3b[sl], q)
        combined = jnp.where(ks == kd, q, jnp.zeros((L,), jnp.float32))
        nbuf[sl] = combined / degs
        return 0

    lax.fori_loop(0, ESTEPS, step, 0)

    plsc.subcore_barrier()
    pltpu.sync_copy(nbuf, accsh.at[didx], add=True)
    plsc.subcore_barrier()

    @pl.when(s == 0)
    def _():
        pltpu.sync_copy(accsh, outp_hbm.at[c])


def _sc_edge_combine(q4, assignments, deg, src, dst):
    zeros = jnp.zeros((N_NODES,), jnp.float32)
    kfn = functools.partial(
        pl.kernel,
        mesh=_sc_mesh(),
        out_type=jax.ShapeDtypeStruct((NC, N_NODES), jnp.float32),
        scratch_types=[
            pltpu.VMEM((EPW,), jnp.int32),
            pltpu.VMEM((EPW,), jnp.int32),
            pltpu.VMEM((EPW,), jnp.float32),
            pltpu.VMEM((EPW,), jnp.float32),
            pltpu.VMEM((EPW,), jnp.float32),
            pltpu.VMEM((EPW,), jnp.float32),
            pltpu.VMEM((EPW,), jnp.int32),
            pltpu.VMEM((EPW,), jnp.int32),
            pltpu.VMEM((EPW,), jnp.float32),
            pltpu.VMEM((EPW,), jnp.float32),
            pltpu.VMEM_SHARED((N_NODES,), jnp.float32),
            pltpu.VMEM_SHARED((N_NODES,), jnp.int32),
            pltpu.VMEM_SHARED((N_NODES,), jnp.float32),
            pltpu.SemaphoreType.DMA,
        ],
    )
    return kfn(_sc_edge_combine_body)(q4, assignments, deg, src, dst, zeros)


def _edge_mlp_body(a_ref, s_ref, W1_ref, b1_ref, W2_ref, b2_ref, eat_ref,
                   cw_ref, Aq_ref, out_ref, q_ref):
    a = a_ref[...]
    b = s_ref[...] - a
    W1 = W1_ref[...]
    h = a * W1[0:1, :] + b * W1[1:2, :] + b1_ref[...]
    h = jnp.maximum(h, 0.0)
    h = jnp.dot(h, W2_ref[...], preferred_element_type=jnp.float32) + b2_ref[...]
    out_ref[...] = jnp.maximum(h, 0.0)
    p = jnp.dot(cw_ref[...], eat_ref[...], preferred_element_type=jnp.float32)
    p1 = p[1:2, :]
    p2 = p[2:3, :]
    l1 = jnp.where(p1 >= 0.0, p1, 0.1 * p1)
    l2 = jnp.where(p2 >= 0.0, p2, 0.1 * p2)
    lmat = jnp.concatenate([p[0:1, :], l1, l2 * l2], axis=0)
    q_ref[...] = jnp.dot(Aq_ref[...], lmat, preferred_element_type=jnp.float32)


def _edge_mlp(a, s, W1, b1, W2, b2, eat, cw, Aq):
    grid = (N_EDGES // EDGE_CHUNK,)
    return pl.pallas_call(
        _edge_mlp_body,
        grid=grid,
        in_specs=[
            pl.BlockSpec((EDGE_CHUNK, 1), lambda i: (i, 0)),
            pl.BlockSpec((EDGE_CHUNK, 1), lambda i: (i, 0)),
            pl.BlockSpec((2, HID), lambda i: (0, 0)),
            pl.BlockSpec((1, HID), lambda i: (0, 0)),
            pl.BlockSpec((HID, HID), lambda i: (0, 0)),
            pl.BlockSpec((1, HID), lambda i: (0, 0)),
            pl.BlockSpec((2, EDGE_CHUNK), lambda i: (0, i)),
            pl.BlockSpec((NUM_POWERS, 2), lambda i: (0, 0)),
            pl.BlockSpec((NUM_KERNELS, NUM_POWERS), lambda i: (0, 0)),
        ],
        out_specs=[
            pl.BlockSpec((EDGE_CHUNK, HID), lambda i: (i, 0)),
            pl.BlockSpec((NUM_KERNELS, EDGE_CHUNK), lambda i: (0, i)),
        ],
        out_shape=[
            jax.ShapeDtypeStruct((N_EDGES, HID), jnp.float32),
            jax.ShapeDtypeStruct((NUM_KERNELS, N_EDGES), jnp.float32),
        ],
    )(a, s, W1, b1, W2, b2, eat, cw, Aq)


def _pool_body(outp_ref, batch_ref, Wfc_ref, bfc_ref, out_ref):
    no = outp_ref[0:1, :] + outp_ref[1:2, :]
    h5 = jnp.maximum(no, 0.0)
    gid = lax.broadcasted_iota(jnp.int32, (NUM_GRAPHS, 1), 0)
    oh = (batch_ref[...] == gid).astype(jnp.float32)
    sums = jnp.sum(oh * h5, axis=1, keepdims=True)
    counts = jnp.sum(oh, axis=1, keepdims=True)
    pooled = sums / jnp.maximum(counts, 1.0)
    out_ref[...] = pooled * Wfc_ref[0, 0] + bfc_ref[0, 0]


def _pool(outp, batch, W_fc, b_fc):
    return pl.pallas_call(
        _pool_body,
        out_shape=jax.ShapeDtypeStruct((NUM_GRAPHS, 1), jnp.float32),
    )(outp, batch.reshape(1, N_NODES), W_fc, b_fc.reshape(1, 1))


def _kmeans_body(x_ref, xt_ref, c0_ref, out_ref):
    X = x_ref[...]
    XT = xt_ref[...]
    x2t = jnp.sum(XT * XT, axis=0, keepdims=True)

    def cond_fn(state):
        _c, i, done = state
        return (i < KMEANS_MAX_ITER) & jnp.logical_not(done)

    def body_fn(state):
        C, i, _done = state
        c2 = jnp.sum(C * C, axis=1, keepdims=True)
        XCt = jnp.dot(C, XT, preferred_element_type=jnp.float32)
        d2 = jnp.maximum(x2t + c2 - 2.0 * XCt, 0.0)
        bv = d2[0:1, :]
        bi = jnp.zeros((1, N_NODES), jnp.int32)
        for j in range(1, NUM_KERNELS):
            take = d2[j:j + 1, :] < bv
            bv = jnp.where(take, d2[j:j + 1, :], bv)
            bi = jnp.where(take, j, bi)
        out_ref[...] = bi
        onehot = (bi == lax.broadcasted_iota(jnp.int32, (NUM_KERNELS, 1), 0)).astype(jnp.float32)
        sums = jnp.dot(onehot, X, preferred_element_type=jnp.float32)
        counts = jnp.sum(onehot, axis=1, keepdims=True)
        new_c = sums / jnp.maximum(counts, 1.0)
        done = jnp.sqrt(jnp.sum((new_c - C) ** 2)) < KMEANS_TOL
        C = jnp.where(done, C, new_c)
        return C, i + 1, done

    C0 = c0_ref[...]
    state = (C0, jnp.int32(0), jnp.bool_(False))
    lax.while_loop(cond_fn, body_fn, state)


def _kmeans(X, key):
    init_idx = jax.random.randint(key, (NUM_KERNELS,), 0, X.shape[0])
    c0 = X[init_idx]
    out = pl.pallas_call(
        _kmeans_body,
        out_shape=jax.ShapeDtypeStruct((1, N_NODES), jnp.int32),
    )(X, X.T, c0)
    return out[0]


def kernel(x, edge_index, edge_attr, batch, W_similar, b_similar, W_ec1, b_ec1,
           W_ec2, b_ec2, conv_w, alpha, W_fc, b_fc):
    src = edge_index[0]
    dst = edge_index[1]

    v = jnp.maximum(x @ (W_similar[:2] + W_similar[2:]) + b_similar, 0.0)

    a, svals, degp = _sc_edge_gather(v[:, 0], src, dst)
    deg = degp[0] + degp[1]

    Aq = alpha[:, :, 0, 0]
    h, q4 = _edge_mlp(a[:, None], svals[:, None], W_ec1, b_ec1.reshape(1, HID),
                      W_ec2, b_ec2.reshape(1, HID), edge_attr.T,
                      conv_w[:, :, 0], Aq)
    x_sim = jnp.zeros((N_NODES, HID), jnp.float32).at[dst].max(
        h, mode="promise_in_bounds", unique_indices=False, indices_are_sorted=False)

    assignments = _kmeans(x_sim, jax.random.key(42))

    outp = _sc_edge_combine(q4.reshape(NUM_KERNELS, NW, EPW), assignments, deg, src, dst)

    return _pool(outp, batch, W_fc, b_fc)


# speedup vs baseline: 1.1760x; 1.0752x over previous
import functools

import jax
import jax.numpy as jnp
from jax import lax
from jax.experimental import pallas as pl
from jax.experimental.pallas import tpu as pltpu
from jax.experimental.pallas import tpu_sc as plsc

N_NODES = 10000
N_EDGES = 320000
NUM_KERNELS = 4
NUM_POWERS = 3
HID = 64
NUM_GRAPHS = 16
KMEANS_MAX_ITER = 300
KMEANS_TOL = 1e-4

EDGE_CHUNK = 6400

NC = 2
NS = 16
L = 16
NW = NC * NS
EPW = N_EDGES // NW
ESTEPS = EPW // L


def _sc_mesh():
    return plsc.VectorSubcoreMesh(core_axis_name="c", subcore_axis_name="s")


def _sc_edge_gather_body(v_hbm, src_hbm, dst_hbm, ones_hbm, zeros_hbm,
                         a_hbm, s_hbm, degp_hbm,
                         sidx, didx, abuf, sbuf, ones, degsh, sem):
    c = lax.axis_index("c")
    s = lax.axis_index("s")
    wid = s * NC + c
    base = wid * EPW
    pltpu.sync_copy(src_hbm.at[pl.ds(base, EPW)], sidx)
    pltpu.sync_copy(dst_hbm.at[pl.ds(base, EPW)], didx)
    pltpu.sync_copy(ones_hbm, ones)

    @pl.when(s == 0)
    def _():
        pltpu.sync_copy(zeros_hbm, degsh)
    plsc.subcore_barrier()

    pltpu.async_copy(v_hbm.at[didx], abuf, sem).wait()
    pltpu.async_copy(v_hbm.at[sidx], sbuf, sem).wait()
    pltpu.sync_copy(abuf, a_hbm.at[pl.ds(base, EPW)])
    pltpu.sync_copy(sbuf, s_hbm.at[pl.ds(base, EPW)])

    pltpu.sync_copy(ones, degsh.at[sidx], add=True)
    plsc.subcore_barrier()

    @pl.when(s == 0)
    def _():
        pltpu.sync_copy(degsh, degp_hbm.at[c])


def _sc_edge_gather(v, src, dst):
    ones = jnp.ones((EPW,), jnp.float32)
    zeros = jnp.zeros((N_NODES,), jnp.float32)
    kfn = functools.partial(
        pl.kernel,
        mesh=_sc_mesh(),
        out_type=[
            jax.ShapeDtypeStruct((N_EDGES,), jnp.float32),
            jax.ShapeDtypeStruct((N_EDGES,), jnp.float32),
            jax.ShapeDtypeStruct((NC, N_NODES), jnp.float32),
        ],
        scratch_types=[
            pltpu.VMEM((EPW,), jnp.int32),
            pltpu.VMEM((EPW,), jnp.int32),
            pltpu.VMEM((EPW,), jnp.float32),
            pltpu.VMEM((EPW,), jnp.float32),
            pltpu.VMEM((EPW,), jnp.float32),
            pltpu.VMEM_SHARED((N_NODES,), jnp.float32),
            pltpu.SemaphoreType.DMA,
        ],
    )
    return kfn(_sc_edge_gather_body)(v, src, dst, ones, zeros)


def _sc_edge_combine_body(q_hbm, assign_hbm, deg_hbm, src_hbm, dst_hbm, zeros_hbm,
                          outp_hbm,
                          sidx, didx, q0b, q1b, q2b, q3b, ksb, kdb, degb, nbuf,
                          accsh, assignsh, degsh, sem):
    c = lax.axis_index("c")
    s = lax.axis_index("s")
    wid = s * NC + c
    base = wid * EPW
    pltpu.sync_copy(src_hbm.at[pl.ds(base, EPW)], sidx)
    pltpu.sync_copy(dst_hbm.at[pl.ds(base, EPW)], didx)
    pltpu.sync_copy(q_hbm.at[0].at[wid], q0b)
    pltpu.sync_copy(q_hbm.at[1].at[wid], q1b)
    pltpu.sync_copy(q_hbm.at[2].at[wid], q2b)
    pltpu.sync_copy(q_hbm.at[3].at[wid], q3b)

    @pl.when(s == 0)
    def _():
        pltpu.sync_copy(zeros_hbm, accsh)
        pltpu.sync_copy(assign_hbm, assignsh)
        pltpu.sync_copy(deg_hbm, degsh)
    plsc.subcore_barrier()

    pltpu.async_copy(assignsh.at[sidx], ksb, sem).wait()
    pltpu.async_copy(assignsh.at[didx], kdb, sem).wait()
    pltpu.async_copy(degsh.at[sidx], degb, sem).wait()

    def step(i, _):
        sl = pl.ds(i * L, L)
        ks = ksb[sl]
        kd = kdb[sl]
        degs = degb[sl]
        q = q0b[sl]
        q = jnp.where(ks == 1, q1b[sl], q)
        q = jnp.where(ks == 2, q2b[sl], q)
        q = jnp.where(ks == 3, q3b[sl], q)
        combined = jnp.where(ks == kd, q, jnp.zeros((L,), jnp.float32))
        nbuf[sl] = combined / degs
        return 0

    lax.fori_loop(0, ESTEPS, step, 0)

    plsc.subcore_barrier()
    pltpu.sync_copy(nbuf, accsh.at[didx], add=True)
    plsc.subcore_barrier()

    @pl.when(s == 0)
    def _():
        pltpu.sync_copy(accsh, outp_hbm.at[c])


def _sc_edge_combine(q4, assignments, deg, src, dst):
    zeros = jnp.zeros((N_NODES,), jnp.float32)
    kfn = functools.partial(
        pl.kernel,
        mesh=_sc_mesh(),
        out_type=jax.ShapeDtypeStruct((NC, N_NODES), jnp.float32),
        scratch_types=[
            pltpu.VMEM((EPW,), jnp.int32),
            pltpu.VMEM((EPW,), jnp.int32),
            pltpu.VMEM((EPW,), jnp.float32),
            pltpu.VMEM((EPW,), jnp.float32),
            pltpu.VMEM((EPW,), jnp.float32),
            pltpu.VMEM((EPW,), jnp.float32),
            pltpu.VMEM((EPW,), jnp.int32),
            pltpu.VMEM((EPW,), jnp.int32),
            pltpu.VMEM((EPW,), jnp.float32),
            pltpu.VMEM((EPW,), jnp.float32),
            pltpu.VMEM_SHARED((N_NODES,), jnp.float32),
            pltpu.VMEM_SHARED((N_NODES,), jnp.int32),
            pltpu.VMEM_SHARED((N_NODES,), jnp.float32),
            pltpu.SemaphoreType.DMA,
        ],
    )
    return kfn(_sc_edge_combine_body)(q4, assignments, deg, src, dst, zeros)


def _edge_mlp_body(a_ref, s_ref, W1_ref, b1_ref, W2_ref, b2_ref, eat_ref,
                   cw_ref, Aq_ref, out_ref, q_ref):
    a = a_ref[...]
    b = s_ref[...] - a
    W1 = W1_ref[...]
    h = a * W1[0:1, :] + b * W1[1:2, :] + b1_ref[...]
    h = jnp.maximum(h, 0.0)
    h = jnp.dot(h, W2_ref[...], preferred_element_type=jnp.float32, precision=lax.Precision.HIGHEST) + b2_ref[...]
    out_ref[...] = jnp.maximum(h, 0.0)
    p = jnp.dot(cw_ref[...], eat_ref[...], preferred_element_type=jnp.float32, precision=lax.Precision.HIGHEST)
    p1 = p[1:2, :]
    p2 = p[2:3, :]
    l1 = jnp.where(p1 >= 0.0, p1, 0.1 * p1)
    l2 = jnp.where(p2 >= 0.0, p2, 0.1 * p2)
    lmat = jnp.concatenate([p[0:1, :], l1, l2 * l2], axis=0)
    q_ref[...] = jnp.dot(Aq_ref[...], lmat, preferred_element_type=jnp.float32, precision=lax.Precision.HIGHEST)


def _edge_mlp(a, s, W1, b1, W2, b2, eat, cw, Aq):
    grid = (N_EDGES // EDGE_CHUNK,)
    return pl.pallas_call(
        _edge_mlp_body,
        grid=grid,
        in_specs=[
            pl.BlockSpec((EDGE_CHUNK, 1), lambda i: (i, 0)),
            pl.BlockSpec((EDGE_CHUNK, 1), lambda i: (i, 0)),
            pl.BlockSpec((2, HID), lambda i: (0, 0)),
            pl.BlockSpec((1, HID), lambda i: (0, 0)),
            pl.BlockSpec((HID, HID), lambda i: (0, 0)),
            pl.BlockSpec((1, HID), lambda i: (0, 0)),
            pl.BlockSpec((2, EDGE_CHUNK), lambda i: (0, i)),
            pl.BlockSpec((NUM_POWERS, 2), lambda i: (0, 0)),
            pl.BlockSpec((NUM_KERNELS, NUM_POWERS), lambda i: (0, 0)),
        ],
        out_specs=[
            pl.BlockSpec((EDGE_CHUNK, HID), lambda i: (i, 0)),
            pl.BlockSpec((NUM_KERNELS, EDGE_CHUNK), lambda i: (0, i)),
        ],
        out_shape=[
            jax.ShapeDtypeStruct((N_EDGES, HID), jnp.float32),
            jax.ShapeDtypeStruct((NUM_KERNELS, N_EDGES), jnp.float32),
        ],
    )(a, s, W1, b1, W2, b2, eat, cw, Aq)


def _pool_body(outp_ref, batch_ref, Wfc_ref, bfc_ref, out_ref):
    no = outp_ref[0:1, :] + outp_ref[1:2, :]
    h5 = jnp.maximum(no, 0.0)
    gid = lax.broadcasted_iota(jnp.int32, (NUM_GRAPHS, 1), 0)
    oh = (batch_ref[...] == gid).astype(jnp.float32)
    sums = jnp.sum(oh * h5, axis=1, keepdims=True)
    counts = jnp.sum(oh, axis=1, keepdims=True)
    pooled = sums / jnp.maximum(counts, 1.0)
    out_ref[...] = pooled * Wfc_ref[0, 0] + bfc_ref[0, 0]


def _pool(outp, batch, W_fc, b_fc):
    return pl.pallas_call(
        _pool_body,
        out_shape=jax.ShapeDtypeStruct((NUM_GRAPHS, 1), jnp.float32),
    )(outp, batch.reshape(1, N_NODES), W_fc, b_fc.reshape(1, 1))


def _kmeans_body(x_ref, xt_ref, c0_ref, out_ref):
    X = x_ref[...]
    XT = xt_ref[...]
    x2t = jnp.sum(XT * XT, axis=0, keepdims=True)

    def cond_fn(state):
        _c, i, done = state
        return (i < KMEANS_MAX_ITER) & jnp.logical_not(done)

    def body_fn(state):
        C, i, _done = state
        c2 = jnp.sum(C * C, axis=1, keepdims=True)
        XCt = jnp.dot(C, XT, preferred_element_type=jnp.float32, precision=lax.Precision.HIGHEST)
        d2 = jnp.maximum(x2t + c2 - 2.0 * XCt, 0.0)
        bv = d2[0:1, :]
        bi = jnp.zeros((1, N_NODES), jnp.int32)
        for j in range(1, NUM_KERNELS):
            take = d2[j:j + 1, :] < bv
            bv = jnp.where(take, d2[j:j + 1, :], bv)
            bi = jnp.where(take, j, bi)
        out_ref[...] = bi
        onehot = (bi == lax.broadcasted_iota(jnp.int32, (NUM_KERNELS, 1), 0)).astype(jnp.float32)
        sums = jnp.dot(onehot, X, preferred_element_type=jnp.float32, precision=lax.Precision.HIGHEST)
        counts = jnp.sum(onehot, axis=1, keepdims=True)
        new_c = sums / jnp.maximum(counts, 1.0)
        done = jnp.sqrt(jnp.sum((new_c - C) ** 2)) < KMEANS_TOL
        C = jnp.where(done, C, new_c)
        return C, i + 1, done

    C0 = c0_ref[...]
    state = (C0, jnp.int32(0), jnp.bool_(False))
    lax.while_loop(cond_fn, body_fn, state)


def _kmeans(X, key):
    init_idx = jax.random.randint(key, (NUM_KERNELS,), 0, X.shape[0])
    c0 = X[init_idx]
    out = pl.pallas_call(
        _kmeans_body,
        out_shape=jax.ShapeDtypeStruct((1, N_NODES), jnp.int32),
    )(X, X.T, c0)
    return out[0]


def kernel(x, edge_index, edge_attr, batch, W_similar, b_similar, W_ec1, b_ec1,
           W_ec2, b_ec2, conv_w, alpha, W_fc, b_fc):
    src = edge_index[0]
    dst = edge_index[1]

    v = jnp.maximum(x @ (W_similar[:2] + W_similar[2:]) + b_similar, 0.0)

    a, svals, degp = _sc_edge_gather(v[:, 0], src, dst)
    deg = degp[0] + degp[1]

    Aq = alpha[:, :, 0, 0]
    h, q4 = _edge_mlp(a[:, None], svals[:, None], W_ec1, b_ec1.reshape(1, HID),
                      W_ec2, b_ec2.reshape(1, HID), edge_attr.T,
                      conv_w[:, :, 0], Aq)
    x_sim = jnp.zeros((N_NODES, HID), jnp.float32).at[dst].max(
        h, mode="promise_in_bounds", unique_indices=False, indices_are_sorted=False)

    assignments = _kmeans(x_sim, jax.random.key(42))

    outp = _sc_edge_combine(q4.reshape(NUM_KERNELS, NW, EPW), assignments, deg, src, dst)

    return _pool(outp, batch, W_fc, b_fc)
